# baseline (speedup 1.0000x reference)
import numpy as np
import jax
import jax.numpy as jnp
from jax import lax
from jax.experimental import pallas as pl
from jax.experimental.pallas import tpu as pltpu
from jax.experimental.pallas import tpu_sc as plsc

_NUM_GROUPS = 256
_GROUP_SIZE = 32
_RADIUS = 0.2
_UPSCALE = 64
_E_IDX = 3

_R2 = np.float32(_RADIUS * _RADIUS)
_HI_BITS = int(np.float32(_R2).view(np.int32)) + 1
_IMAX = np.int32(2**31 - 1)
_NW = 32
_RPW = (4 * _NUM_GROUPS) // _NW


def _fps_kernel(pt_ref, valid_ref, centers_ref, md_ref):
    B, _, N = pt_ref.shape
    x = pt_ref[:, 0, :]
    y = pt_ref[:, 1, :]
    z = pt_ref[:, 2, :]
    e = pt_ref[:, 3, :]
    valid = valid_ref[:, 0, :] != 0
    lane = jax.lax.broadcasted_iota(jnp.int32, (B, N), 1)
    inf = jnp.float32(jnp.inf)
    md_ref[...] = jnp.where(valid, inf, -inf)

    def pick(col):
        onehot = lane == col
        sel = lambda v: jnp.sum(jnp.where(onehot, v, 0.0), axis=1, keepdims=True)
        return sel(x), sel(y), sel(z), sel(e)

    def emit(i, coords):
        lx, ly, lz, le = coords
        row = jnp.concatenate([lx, ly, lz, le], axis=1).reshape(B, 1, 4)
        centers_ref[:, pl.ds(i, 1), :] = row

    first = pick(jnp.zeros((B, 1), jnp.int32))
    emit(0, first)

    def step(i, coords):
        lx, ly, lz, _ = coords
        dx = x - lx
        dy = y - ly
        dz = z - lz
        d = dx * dx + dy * dy + dz * dz
        md = jnp.minimum(md_ref[...], jnp.where(valid, d, -inf))
        md_ref[...] = md
        m = jnp.max(md, axis=1, keepdims=True)
        nxt = jnp.min(jnp.where(md == m, lane, N), axis=1, keepdims=True)
        coords = pick(nxt)
        emit(i, coords)
        return coords

    jax.lax.fori_loop(1, _NUM_GROUPS, step, first)


def _fps_centers(points_t, valid):
    B, _, N = points_t.shape
    return pl.pallas_call(
        _fps_kernel,
        out_shape=jax.ShapeDtypeStruct((B, _NUM_GROUPS, 4), jnp.float32),
        scratch_shapes=[pltpu.VMEM((B, N), jnp.float32)],
    )(points_t, valid)


def _iota16():
    return lax.broadcasted_iota(jnp.int32, (16,), 0)


def _splat(x):
    return jnp.broadcast_to(x, (16,))


def _select_kernel(planes, centers, lengths, groups_out, mask_out,
                   px_v, py_v, pz_v, pe_v, cand_db, cand_ix, pool_db, pool_ix,
                   outidx_v, outrow_v, maskrow_v, centers_v, len_v):
    wid = lax.axis_index("s") * 2 + lax.axis_index("c")
    batch = wid // (_NUM_GROUPS // _RPW)
    iota = _iota16()

    for ch, pv in zip(range(4), (px_v, py_v, pz_v, pe_v)):
        pltpu.sync_copy(planes.at[batch, ch], pv)
    pltpu.sync_copy(centers.at[pl.ds(wid * _RPW, _RPW)], centers_v)
    pltpu.sync_copy(lengths, len_v)
    lenb = plsc.load_gather(len_v, [_splat(batch)])

    def row_body(r, _):
        row = wid * _RPW + r
        cx = plsc.load_gather(centers_v, [_splat(r), _splat(jnp.int32(0))])
        cy = plsc.load_gather(centers_v, [_splat(r), _splat(jnp.int32(1))])
        cz = plsc.load_gather(centers_v, [_splat(r), _splat(jnp.int32(2))])

        def compact(c, off_s):
            sl = pl.ds(c * 16, 16)
            dx = cx - px_v[sl]
            dy = cy - py_v[sl]
            dz = cz - pz_v[sl]
            v = dx * dx + dy * dy + dz * dz
            gidx = c * 16 + iota
            m = (v <= _R2) & (gidx < lenb)
            mi = m.astype(jnp.int32)
            incl = plsc.cumsum(mi)
            dest = _splat(off_s) + incl - mi
            plsc.store_scatter(cand_db, [dest], plsc.bitcast(v, jnp.int32), mask=m)
            plsc.store_scatter(cand_ix, [dest], gidx, mask=m)
            return off_s + jnp.sum(mi)

        cnt_s = lax.fori_loop(0, 512, compact, jnp.int32(0))
        cnt = _splat(cnt_s)
        nchunks = (cnt_s + 15) // 16
        kwant_s = jnp.minimum(cnt_s, _UPSCALE)

        def count_le(mid_s):
            mid = _splat(mid_s)

            def cc(c, acc_s):
                vb = cand_db[pl.ds(c * 16, 16)]
                inb = (c * 16 + iota) < cnt
                return acc_s + jnp.sum(((vb <= mid) & inb).astype(jnp.int32))

            return lax.fori_loop(0, nchunks, cc, jnp.int32(0))

        def bs(_, lohi):
            lo, hi = lohi
            mid = (lo + hi) // 2
            ge = count_le(mid) >= kwant_s
            return (jnp.where(ge, lo, mid + 1), jnp.where(ge, mid, hi))

        t_s, _hi = lax.fori_loop(0, 31, bs,
                                 (jnp.int32(0), jnp.int32(_HI_BITS)))
        t = _splat(t_s)
        n_lt_s = count_le(t_s - 1)
        need_eq = _splat(kwant_s - n_lt_s)

        for j in range(5):
            pool_db[pl.ds(16 * j, 16)] = _splat(_IMAX)
            pool_ix[pl.ds(16 * j, 16)] = _splat(jnp.int32(0))

        def pbuild(c, carry):
            poff_s, eqs_s = carry
            vb = cand_db[pl.ds(c * 16, 16)]
            vi = cand_ix[pl.ds(c * 16, 16)]
            inb = (c * 16 + iota) < cnt
            is_lt = (vb < t) & inb
            is_eq = (vb == t) & inb
            eqi = is_eq.astype(jnp.int32)
            eq_excl = _splat(eqs_s) + plsc.cumsum(eqi) - eqi
            take = is_lt | (is_eq & (eq_excl < need_eq))
            ti = take.astype(jnp.int32)
            dest = _splat(poff_s) + plsc.cumsum(ti) - ti
            plsc.store_scatter(pool_db, [dest], vb, mask=take)
            plsc.store_scatter(pool_ix, [dest], vi, mask=take)
            return (poff_s + jnp.sum(ti), eqs_s + jnp.sum(eqi))

        lax.fori_loop(0, nchunks, pbuild, (jnp.int32(0), jnp.int32(0)))
        pool_n = _splat(kwant_s)

        peb, pdb, pix = [], [], []
        for j in range(4):
            inb = (16 * j + iota) < pool_n
            pj = pool_ix[pl.ds(16 * j, 16)]
            ev = plsc.load_gather(pe_v, [pj])
            peb.append(jnp.where(inb, plsc.bitcast(ev, jnp.int32),
                                 _splat(jnp.int32(-1))))
            pdb.append(pool_db[pl.ds(16 * j, 16)])
            pix.append(pj)

        def extract(k, eb):
            eb = list(eb)
            m_s = jnp.max(jnp.maximum(jnp.maximum(eb[0], eb[1]),
                                      jnp.maximum(eb[2], eb[3])))
            me = _splat(m_s)
            tie = [eb[j] == me for j in range(4)]
            dmv = [jnp.where(tie[j], pdb[j], _splat(_IMAX)) for j in range(4)]
            d_s = jnp.min(jnp.minimum(jnp.minimum(dmv[0], dmv[1]),
                                      jnp.minimum(dmv[2], dmv[3])))
            dm = _splat(d_s)
            tie2 = [tie[j] & (pdb[j] == dm) for j in range(4)]
            imv = [jnp.where(tie2[j], pix[j], _splat(_IMAX)) for j in range(4)]
            i_s = jnp.min(jnp.minimum(jnp.minimum(imv[0], imv[1]),
                                      jnp.minimum(imv[2], imv[3])))
            im = _splat(i_s)
            chosen = [tie2[j] & (pix[j] == im) for j in range(4)]
            eb = [jnp.where(chosen[j], _splat(jnp.int32(-1)), eb[j])
                  for j in range(4)]
            plsc.store_scatter(outidx_v, [_splat(k)], im, mask=iota == 0)
            return tuple(eb)

        lax.fori_loop(0, _GROUP_SIZE, extract, tuple(peb))

        inv_r = jnp.float32(1.0 / _RADIUS)
        nvalid = _splat(jnp.minimum(kwant_s, _GROUP_SIZE))
        for j in range(2):
            oi = outidx_v[pl.ds(16 * j, 16)]
            vk = (16 * j + iota) < nvalid
            px = plsc.load_gather(px_v, [oi])
            py = plsc.load_gather(py_v, [oi])
            pz = plsc.load_gather(pz_v, [oi])
            pe = plsc.load_gather(pe_v, [oi])
            zero = _splat(jnp.float32(0.0))
            gx = jnp.where(vk, (px - cx) * inv_r, zero)
            gy = jnp.where(vk, (py - cy) * inv_r, zero)
            gz = jnp.where(vk, (pz - cz) * inv_r, zero)
            ge = jnp.where(vk, pe, zero)
            base = (16 * j + iota) * 4
            plsc.store_scatter(outrow_v, [base], gx)
            plsc.store_scatter(outrow_v, [base + 1], gy)
            plsc.store_scatter(outrow_v, [base + 2], gz)
            plsc.store_scatter(outrow_v, [base + 3], ge)
            maskrow_v[pl.ds(16 * j, 16)] = vk.astype(jnp.int32)
        pltpu.sync_copy(outrow_v, groups_out.at[row])
        pltpu.sync_copy(maskrow_v, mask_out.at[row])
        return 0

    lax.fori_loop(0, _RPW, row_body, 0)


def _sc_select(planes, centers_flat, lengths_pad):
    _, _, N = planes.shape
    BG = centers_flat.shape[0]
    mesh = plsc.VectorSubcoreMesh(core_axis_name="c", subcore_axis_name="s")
    f = pl.kernel(
        _select_kernel,
        compiler_params=pltpu.CompilerParams(needs_layout_passes=False),
        out_type=[
            jax.ShapeDtypeStruct((BG, 4 * _GROUP_SIZE), jnp.float32),
            jax.ShapeDtypeStruct((BG, _GROUP_SIZE), jnp.int32),
        ],
        mesh=mesh,
        scratch_types=[
            pltpu.VMEM((N,), jnp.float32),
            pltpu.VMEM((N,), jnp.float32),
            pltpu.VMEM((N,), jnp.float32),
            pltpu.VMEM((N,), jnp.float32),
            pltpu.VMEM((N,), jnp.int32),
            pltpu.VMEM((N,), jnp.int32),
            pltpu.VMEM((80,), jnp.int32),
            pltpu.VMEM((80,), jnp.int32),
            pltpu.VMEM((_GROUP_SIZE,), jnp.int32),
            pltpu.VMEM((4 * _GROUP_SIZE,), jnp.float32),
            pltpu.VMEM((_GROUP_SIZE,), jnp.int32),
            pltpu.VMEM((_RPW, 4), jnp.float32),
            pltpu.VMEM((8,), jnp.int32),
        ],
    )
    return f(planes, centers_flat, lengths_pad)


def kernel(points, lengths):
    B, N, C = points.shape
    points_t = points.transpose(0, 2, 1)
    valid = (jnp.arange(N)[None, :] < lengths[:, None]).astype(jnp.int32)
    valid = valid.reshape(B, 1, N)
    centers = _fps_centers(points_t, valid)
    lengths_pad = jnp.concatenate([lengths, jnp.zeros((4,), jnp.int32)])
    groups_flat, mask_flat = _sc_select(
        points_t, centers.reshape(B * _NUM_GROUPS, 4), lengths_pad)
    groups = groups_flat.reshape(B, _NUM_GROUPS, _GROUP_SIZE, 4)
    mask = mask_flat.reshape(B, _NUM_GROUPS, _GROUP_SIZE) != 0
    return groups, centers, mask


# speedup vs baseline: 1.0243x; 1.0243x over previous
import numpy as np
import jax
import jax.numpy as jnp
from jax import lax
from jax.experimental import pallas as pl
from jax.experimental.pallas import tpu as pltpu
from jax.experimental.pallas import tpu_sc as plsc

_NUM_GROUPS = 256
_GROUP_SIZE = 32
_RADIUS = 0.2
_UPSCALE = 64
_E_IDX = 3

_R2 = np.float32(_RADIUS * _RADIUS)
_HI_BITS = int(np.float32(_R2).view(np.int32)) + 1
_IMAX = np.int32(2**31 - 1)
_NW = 32
_RPW = (4 * _NUM_GROUPS) // _NW


def _fps_kernel(pt_ref, valid_ref, centers_ref, md_ref):
    B, _, N = pt_ref.shape
    x = pt_ref[:, 0, :]
    y = pt_ref[:, 1, :]
    z = pt_ref[:, 2, :]
    e = pt_ref[:, 3, :]
    valid = valid_ref[:, 0, :] != 0
    lane = jax.lax.broadcasted_iota(jnp.int32, (B, N), 1)
    inf = jnp.float32(jnp.inf)
    md_ref[...] = jnp.where(valid, inf, -inf)

    def pick(col):
        onehot = lane == col
        sel = lambda v: jnp.sum(jnp.where(onehot, v, 0.0), axis=1, keepdims=True)
        return sel(x), sel(y), sel(z), sel(e)

    def emit(i, coords):
        lx, ly, lz, le = coords
        row = jnp.concatenate([lx, ly, lz, le], axis=1).reshape(B, 1, 4)
        centers_ref[:, pl.ds(i, 1), :] = row

    first = pick(jnp.zeros((B, 1), jnp.int32))
    emit(0, first)

    def step(i, coords):
        lx, ly, lz, _ = coords
        dx = x - lx
        dy = y - ly
        dz = z - lz
        d = dx * dx + dy * dy + dz * dz
        md = jnp.minimum(md_ref[...], jnp.where(valid, d, -inf))
        md_ref[...] = md
        m = jnp.max(md, axis=1, keepdims=True)
        nxt = jnp.min(jnp.where(md == m, lane, N), axis=1, keepdims=True)
        coords = pick(nxt)
        emit(i, coords)
        return coords

    jax.lax.fori_loop(1, _NUM_GROUPS, step, first)


def _fps_centers(points_t, valid):
    B, _, N = points_t.shape
    return pl.pallas_call(
        _fps_kernel,
        out_shape=jax.ShapeDtypeStruct((B, _NUM_GROUPS, 4), jnp.float32),
        scratch_shapes=[pltpu.VMEM((B, N), jnp.float32)],
    )(points_t, valid)


def _iota16():
    return lax.broadcasted_iota(jnp.int32, (16,), 0)


def _splat(x):
    return jnp.broadcast_to(x, (16,))


def _select_kernel(planes, centers, lengths, groups_out, mask_out,
                   px_v, py_v, pz_v, pe_v, cand_db, cand_ix, pool_db, pool_ix,
                   outidx_v, outrow_v, maskrow_v, centers_v, len_v):
    wid = lax.axis_index("s") * 2 + lax.axis_index("c")
    batch = wid // (_NUM_GROUPS // _RPW)
    iota = _iota16()

    for ch, pv in zip(range(4), (px_v, py_v, pz_v, pe_v)):
        pltpu.sync_copy(planes.at[batch, ch], pv)
    pltpu.sync_copy(centers.at[pl.ds(wid * _RPW, _RPW)], centers_v)
    pltpu.sync_copy(lengths, len_v)
    lenb = plsc.load_gather(len_v, [_splat(batch)])

    def row_body(r, _):
        row = wid * _RPW + r
        cx = plsc.load_gather(centers_v, [_splat(r), _splat(jnp.int32(0))])
        cy = plsc.load_gather(centers_v, [_splat(r), _splat(jnp.int32(1))])
        cz = plsc.load_gather(centers_v, [_splat(r), _splat(jnp.int32(2))])

        def compact(c, off_s):
            sl = pl.ds(c * 16, 16)
            dx = cx - px_v[sl]
            dy = cy - py_v[sl]
            dz = cz - pz_v[sl]
            v = dx * dx + dy * dy + dz * dz
            gidx = c * 16 + iota
            m = (v <= _R2) & (gidx < lenb)
            mi = m.astype(jnp.int32)
            incl = plsc.cumsum(mi)
            dest = _splat(off_s) + incl - mi
            plsc.store_scatter(cand_db, [dest], plsc.bitcast(v, jnp.int32), mask=m)
            plsc.store_scatter(cand_ix, [dest], gidx, mask=m)
            return off_s + incl[15]

        cnt_s = lax.fori_loop(0, 512, compact, jnp.int32(0))
        cnt = _splat(cnt_s)
        nchunks = (cnt_s + 15) // 16
        kwant_s = jnp.minimum(cnt_s, _UPSCALE)

        def count_le(mid_s):
            mid = _splat(mid_s)

            def cc(c, acc_v):
                vb = cand_db[pl.ds(c * 16, 16)]
                inb = (c * 16 + iota) < cnt
                return acc_v + ((vb <= mid) & inb).astype(jnp.int32)

            acc = lax.fori_loop(0, nchunks, cc, _splat(jnp.int32(0)))
            return jnp.sum(acc)

        def bs(_, lohi):
            lo, hi = lohi
            mid = (lo + hi) // 2
            ge = count_le(mid) >= kwant_s
            return (jnp.where(ge, lo, mid + 1), jnp.where(ge, mid, hi))

        t_s, _hi = lax.fori_loop(0, 31, bs,
                                 (jnp.int32(0), jnp.int32(_HI_BITS)))
        t = _splat(t_s)
        n_lt_s = count_le(t_s - 1)
        need_eq = _splat(kwant_s - n_lt_s)

        for j in range(5):
            pool_db[pl.ds(16 * j, 16)] = _splat(_IMAX)
            pool_ix[pl.ds(16 * j, 16)] = _splat(jnp.int32(0))

        def pbuild(c, carry):
            poff_s, eqs_s = carry
            vb = cand_db[pl.ds(c * 16, 16)]
            vi = cand_ix[pl.ds(c * 16, 16)]
            inb = (c * 16 + iota) < cnt
            is_lt = (vb < t) & inb
            is_eq = (vb == t) & inb
            eqi = is_eq.astype(jnp.int32)
            eq_excl = _splat(eqs_s) + plsc.cumsum(eqi) - eqi
            take = is_lt | (is_eq & (eq_excl < need_eq))
            ti = take.astype(jnp.int32)
            dest = _splat(poff_s) + plsc.cumsum(ti) - ti
            plsc.store_scatter(pool_db, [dest], vb, mask=take)
            plsc.store_scatter(pool_ix, [dest], vi, mask=take)
            return (poff_s + jnp.sum(ti), eqs_s + jnp.sum(eqi))

        lax.fori_loop(0, nchunks, pbuild, (jnp.int32(0), jnp.int32(0)))
        pool_n = _splat(kwant_s)

        peb, pdb, pix = [], [], []
        for j in range(4):
            inb = (16 * j + iota) < pool_n
            pj = pool_ix[pl.ds(16 * j, 16)]
            ev = plsc.load_gather(pe_v, [pj])
            peb.append(jnp.where(inb, plsc.bitcast(ev, jnp.int32),
                                 _splat(jnp.int32(-1))))
            pdb.append(pool_db[pl.ds(16 * j, 16)])
            pix.append(pj)

        def extract(k, eb):
            eb = list(eb)
            m_s = jnp.max(jnp.maximum(jnp.maximum(eb[0], eb[1]),
                                      jnp.maximum(eb[2], eb[3])))
            me = _splat(m_s)
            tie = [eb[j] == me for j in range(4)]
            dmv = [jnp.where(tie[j], pdb[j], _splat(_IMAX)) for j in range(4)]
            d_s = jnp.min(jnp.minimum(jnp.minimum(dmv[0], dmv[1]),
                                      jnp.minimum(dmv[2], dmv[3])))
            dm = _splat(d_s)
            tie2 = [tie[j] & (pdb[j] == dm) for j in range(4)]
            imv = [jnp.where(tie2[j], pix[j], _splat(_IMAX)) for j in range(4)]
            i_s = jnp.min(jnp.minimum(jnp.minimum(imv[0], imv[1]),
                                      jnp.minimum(imv[2], imv[3])))
            im = _splat(i_s)
            chosen = [tie2[j] & (pix[j] == im) for j in range(4)]
            eb = [jnp.where(chosen[j], _splat(jnp.int32(-1)), eb[j])
                  for j in range(4)]
            plsc.store_scatter(outidx_v, [_splat(k)], im, mask=iota == 0)
            return tuple(eb)

        lax.fori_loop(0, _GROUP_SIZE, extract, tuple(peb))

        inv_r = jnp.float32(1.0 / _RADIUS)
        nvalid = _splat(jnp.minimum(kwant_s, _GROUP_SIZE))
        for j in range(2):
            oi = outidx_v[pl.ds(16 * j, 16)]
            vk = (16 * j + iota) < nvalid
            px = plsc.load_gather(px_v, [oi])
            py = plsc.load_gather(py_v, [oi])
            pz = plsc.load_gather(pz_v, [oi])
            pe = plsc.load_gather(pe_v, [oi])
            zero = _splat(jnp.float32(0.0))
            gx = jnp.where(vk, (px - cx) * inv_r, zero)
            gy = jnp.where(vk, (py - cy) * inv_r, zero)
            gz = jnp.where(vk, (pz - cz) * inv_r, zero)
            ge = jnp.where(vk, pe, zero)
            base = (16 * j + iota) * 4
            plsc.store_scatter(outrow_v, [base], gx)
            plsc.store_scatter(outrow_v, [base + 1], gy)
            plsc.store_scatter(outrow_v, [base + 2], gz)
            plsc.store_scatter(outrow_v, [base + 3], ge)
            maskrow_v[pl.ds(16 * j, 16)] = vk.astype(jnp.int32)
        pltpu.sync_copy(outrow_v, groups_out.at[row])
        pltpu.sync_copy(maskrow_v, mask_out.at[row])
        return 0

    lax.fori_loop(0, _RPW, row_body, 0)


def _sc_select(planes, centers_flat, lengths_pad):
    _, _, N = planes.shape
    BG = centers_flat.shape[0]
    mesh = plsc.VectorSubcoreMesh(core_axis_name="c", subcore_axis_name="s")
    f = pl.kernel(
        _select_kernel,
        compiler_params=pltpu.CompilerParams(needs_layout_passes=False),
        out_type=[
            jax.ShapeDtypeStruct((BG, 4 * _GROUP_SIZE), jnp.float32),
            jax.ShapeDtypeStruct((BG, _GROUP_SIZE), jnp.int32),
        ],
        mesh=mesh,
        scratch_types=[
            pltpu.VMEM((N,), jnp.float32),
            pltpu.VMEM((N,), jnp.float32),
            pltpu.VMEM((N,), jnp.float32),
            pltpu.VMEM((N,), jnp.float32),
            pltpu.VMEM((N,), jnp.int32),
            pltpu.VMEM((N,), jnp.int32),
            pltpu.VMEM((80,), jnp.int32),
            pltpu.VMEM((80,), jnp.int32),
            pltpu.VMEM((_GROUP_SIZE,), jnp.int32),
            pltpu.VMEM((4 * _GROUP_SIZE,), jnp.float32),
            pltpu.VMEM((_GROUP_SIZE,), jnp.int32),
            pltpu.VMEM((_RPW, 4), jnp.float32),
            pltpu.VMEM((8,), jnp.int32),
        ],
    )
    return f(planes, centers_flat, lengths_pad)


def kernel(points, lengths):
    B, N, C = points.shape
    points_t = points.transpose(0, 2, 1)
    valid = (jnp.arange(N)[None, :] < lengths[:, None]).astype(jnp.int32)
    valid = valid.reshape(B, 1, N)
    centers = _fps_centers(points_t, valid)
    lengths_pad = jnp.concatenate([lengths, jnp.zeros((4,), jnp.int32)])
    groups_flat, mask_flat = _sc_select(
        points_t, centers.reshape(B * _NUM_GROUPS, 4), lengths_pad)
    groups = groups_flat.reshape(B, _NUM_GROUPS, _GROUP_SIZE, 4)
    mask = mask_flat.reshape(B, _NUM_GROUPS, _GROUP_SIZE) != 0
    return groups, centers, mask


# speedup vs baseline: 1.7476x; 1.7061x over previous
import numpy as np
import jax
import jax.numpy as jnp
from jax import lax
from jax.experimental import pallas as pl
from jax.experimental.pallas import tpu as pltpu
from jax.experimental.pallas import tpu_sc as plsc

_NUM_GROUPS = 256
_GROUP_SIZE = 32
_RADIUS = 0.2
_UPSCALE = 64
_E_IDX = 3

_R2 = np.float32(_RADIUS * _RADIUS)
_HI_BITS = int(np.float32(_R2).view(np.int32)) + 1
_IMAX = np.int32(2**31 - 1)
_NW = 32
_RPW = (4 * _NUM_GROUPS) // _NW


def _fps_kernel(pt_ref, valid_ref, centers_ref, md_ref):
    B, _, N = pt_ref.shape
    x = pt_ref[:, 0, :]
    y = pt_ref[:, 1, :]
    z = pt_ref[:, 2, :]
    e = pt_ref[:, 3, :]
    valid = valid_ref[:, 0, :] != 0
    lane = jax.lax.broadcasted_iota(jnp.int32, (B, N), 1)
    inf = jnp.float32(jnp.inf)
    md_ref[...] = jnp.where(valid, inf, -inf)

    def pick(col):
        onehot = lane == col
        sel = lambda v: jnp.sum(jnp.where(onehot, v, 0.0), axis=1, keepdims=True)
        return sel(x), sel(y), sel(z), sel(e)

    def emit(i, coords):
        lx, ly, lz, le = coords
        row = jnp.concatenate([lx, ly, lz, le], axis=1).reshape(B, 1, 4)
        centers_ref[:, pl.ds(i, 1), :] = row

    first = pick(jnp.zeros((B, 1), jnp.int32))
    emit(0, first)

    def step(i, coords):
        lx, ly, lz, _ = coords
        dx = x - lx
        dy = y - ly
        dz = z - lz
        d = dx * dx + dy * dy + dz * dz
        md = jnp.minimum(md_ref[...], jnp.where(valid, d, -inf))
        md_ref[...] = md
        m = jnp.max(md, axis=1, keepdims=True)
        nxt = jnp.min(jnp.where(md == m, lane, N), axis=1, keepdims=True)
        coords = pick(nxt)
        emit(i, coords)
        return coords

    jax.lax.fori_loop(1, _NUM_GROUPS, step, first)


def _fps_centers(points_t, valid):
    B, _, N = points_t.shape
    return pl.pallas_call(
        _fps_kernel,
        out_shape=jax.ShapeDtypeStruct((B, _NUM_GROUPS, 4), jnp.float32),
        scratch_shapes=[pltpu.VMEM((B, N), jnp.float32)],
    )(points_t, valid)


def _iota16():
    return lax.broadcasted_iota(jnp.int32, (16,), 0)


def _splat(x):
    return jnp.broadcast_to(x, (16,))


def _select_kernel(planes, centers, lengths, groups_out, mask_out,
                   px_v, py_v, pz_v, pe_v, cand_db, cand_ix, pool_db, pool_ix,
                   outidx_v, outrow_v, maskrow_v, centers_v, len_v):
    wid = lax.axis_index("s") * 2 + lax.axis_index("c")
    batch = wid // (_NUM_GROUPS // _RPW)
    iota = _iota16()

    for ch, pv in zip(range(4), (px_v, py_v, pz_v, pe_v)):
        pltpu.sync_copy(planes.at[batch, ch], pv)
    pltpu.sync_copy(centers.at[pl.ds(wid * _RPW, _RPW)], centers_v)
    pltpu.sync_copy(lengths, len_v)
    lenb = plsc.load_gather(len_v, [_splat(batch)])

    def row_body(r, _):
        row = wid * _RPW + r
        cx = plsc.load_gather(centers_v, [_splat(r), _splat(jnp.int32(0))])
        cy = plsc.load_gather(centers_v, [_splat(r), _splat(jnp.int32(1))])
        cz = plsc.load_gather(centers_v, [_splat(r), _splat(jnp.int32(2))])

        def compact(c, off_s):
            sl = pl.ds(c * 16, 16)
            dx = cx - px_v[sl]
            dy = cy - py_v[sl]
            dz = cz - pz_v[sl]
            v = dx * dx + dy * dy + dz * dz
            gidx = c * 16 + iota
            m = (v <= _R2) & (gidx < lenb)
            mi = m.astype(jnp.int32)
            incl = plsc.cumsum(mi)
            dest = _splat(off_s) + incl - mi
            plsc.store_scatter(cand_db, [dest], plsc.bitcast(v, jnp.int32), mask=m)
            plsc.store_scatter(cand_ix, [dest], gidx, mask=m)
            return off_s + incl[15]

        cnt_s = plsc.parallel_loop(0, 512, unroll=4, carry=jnp.int32(0))(
            lambda c, off_s: compact(c, off_s))
        cnt = _splat(cnt_s)
        nchunks = (cnt_s + 15) // 16
        kwant_s = jnp.minimum(cnt_s, _UPSCALE)

        def count_le(mid_s):
            mid = _splat(mid_s)

            def cc(c, acc_v):
                vb = cand_db[pl.ds(c * 16, 16)]
                inb = (c * 16 + iota) < cnt
                return acc_v + ((vb <= mid) & inb).astype(jnp.int32)

            acc = plsc.parallel_loop(0, nchunks, unroll=4,
                                     carry=_splat(jnp.int32(0)))(cc)
            return jnp.sum(acc)

        def bs(_, lohi):
            lo, hi = lohi
            mid = (lo + hi) // 2
            ge = count_le(mid) >= kwant_s
            return (jnp.where(ge, lo, mid + 1), jnp.where(ge, mid, hi))

        t_s, _hi = lax.fori_loop(0, 31, bs,
                                 (jnp.int32(0), jnp.int32(_HI_BITS)))
        t = _splat(t_s)
        n_lt_s = count_le(t_s - 1)
        need_eq = _splat(kwant_s - n_lt_s)

        for j in range(5):
            pool_db[pl.ds(16 * j, 16)] = _splat(_IMAX)
            pool_ix[pl.ds(16 * j, 16)] = _splat(jnp.int32(0))

        def pbuild(c, carry):
            poff_s, eqs_s = carry
            vb = cand_db[pl.ds(c * 16, 16)]
            vi = cand_ix[pl.ds(c * 16, 16)]
            inb = (c * 16 + iota) < cnt
            is_lt = (vb < t) & inb
            is_eq = (vb == t) & inb
            eqi = is_eq.astype(jnp.int32)
            eq_excl = _splat(eqs_s) + plsc.cumsum(eqi) - eqi
            take = is_lt | (is_eq & (eq_excl < need_eq))
            ti = take.astype(jnp.int32)
            dest = _splat(poff_s) + plsc.cumsum(ti) - ti
            plsc.store_scatter(pool_db, [dest], vb, mask=take)
            plsc.store_scatter(pool_ix, [dest], vi, mask=take)
            return (poff_s + jnp.sum(ti), eqs_s + jnp.sum(eqi))

        lax.fori_loop(0, nchunks, pbuild, (jnp.int32(0), jnp.int32(0)))
        pool_n = _splat(kwant_s)

        peb, pdb, pix = [], [], []
        for j in range(4):
            inb = (16 * j + iota) < pool_n
            pj = pool_ix[pl.ds(16 * j, 16)]
            ev = plsc.load_gather(pe_v, [pj])
            peb.append(jnp.where(inb, plsc.bitcast(ev, jnp.int32),
                                 _splat(jnp.int32(-1))))
            pdb.append(pool_db[pl.ds(16 * j, 16)])
            pix.append(pj)

        def extract(k, eb):
            eb = list(eb)
            m_s = jnp.max(jnp.maximum(jnp.maximum(eb[0], eb[1]),
                                      jnp.maximum(eb[2], eb[3])))
            me = _splat(m_s)
            tie = [eb[j] == me for j in range(4)]
            dmv = [jnp.where(tie[j], pdb[j], _splat(_IMAX)) for j in range(4)]
            d_s = jnp.min(jnp.minimum(jnp.minimum(dmv[0], dmv[1]),
                                      jnp.minimum(dmv[2], dmv[3])))
            dm = _splat(d_s)
            tie2 = [tie[j] & (pdb[j] == dm) for j in range(4)]
            imv = [jnp.where(tie2[j], pix[j], _splat(_IMAX)) for j in range(4)]
            i_s = jnp.min(jnp.minimum(jnp.minimum(imv[0], imv[1]),
                                      jnp.minimum(imv[2], imv[3])))
            im = _splat(i_s)
            chosen = [tie2[j] & (pix[j] == im) for j in range(4)]
            eb = [jnp.where(chosen[j], _splat(jnp.int32(-1)), eb[j])
                  for j in range(4)]
            plsc.store_scatter(outidx_v, [_splat(k)], im, mask=iota == 0)
            return tuple(eb)

        lax.fori_loop(0, _GROUP_SIZE, extract, tuple(peb))

        inv_r = jnp.float32(1.0 / _RADIUS)
        nvalid = _splat(jnp.minimum(kwant_s, _GROUP_SIZE))
        for j in range(2):
            oi = outidx_v[pl.ds(16 * j, 16)]
            vk = (16 * j + iota) < nvalid
            px = plsc.load_gather(px_v, [oi])
            py = plsc.load_gather(py_v, [oi])
            pz = plsc.load_gather(pz_v, [oi])
            pe = plsc.load_gather(pe_v, [oi])
            zero = _splat(jnp.float32(0.0))
            gx = jnp.where(vk, (px - cx) * inv_r, zero)
            gy = jnp.where(vk, (py - cy) * inv_r, zero)
            gz = jnp.where(vk, (pz - cz) * inv_r, zero)
            ge = jnp.where(vk, pe, zero)
            base = (16 * j + iota) * 4
            plsc.store_scatter(outrow_v, [base], gx)
            plsc.store_scatter(outrow_v, [base + 1], gy)
            plsc.store_scatter(outrow_v, [base + 2], gz)
            plsc.store_scatter(outrow_v, [base + 3], ge)
            maskrow_v[pl.ds(16 * j, 16)] = vk.astype(jnp.int32)
        pltpu.sync_copy(outrow_v, groups_out.at[row])
        pltpu.sync_copy(maskrow_v, mask_out.at[row])
        return 0

    lax.fori_loop(0, _RPW, row_body, 0)


def _sc_select(planes, centers_flat, lengths_pad):
    _, _, N = planes.shape
    BG = centers_flat.shape[0]
    mesh = plsc.VectorSubcoreMesh(core_axis_name="c", subcore_axis_name="s")
    f = pl.kernel(
        _select_kernel,
        compiler_params=pltpu.CompilerParams(needs_layout_passes=False),
        out_type=[
            jax.ShapeDtypeStruct((BG, 4 * _GROUP_SIZE), jnp.float32),
            jax.ShapeDtypeStruct((BG, _GROUP_SIZE), jnp.int32),
        ],
        mesh=mesh,
        scratch_types=[
            pltpu.VMEM((N,), jnp.float32),
            pltpu.VMEM((N,), jnp.float32),
            pltpu.VMEM((N,), jnp.float32),
            pltpu.VMEM((N,), jnp.float32),
            pltpu.VMEM((N,), jnp.int32),
            pltpu.VMEM((N,), jnp.int32),
            pltpu.VMEM((80,), jnp.int32),
            pltpu.VMEM((80,), jnp.int32),
            pltpu.VMEM((_GROUP_SIZE,), jnp.int32),
            pltpu.VMEM((4 * _GROUP_SIZE,), jnp.float32),
            pltpu.VMEM((_GROUP_SIZE,), jnp.int32),
            pltpu.VMEM((_RPW, 4), jnp.float32),
            pltpu.VMEM((8,), jnp.int32),
        ],
    )
    return f(planes, centers_flat, lengths_pad)


def kernel(points, lengths):
    B, N, C = points.shape
    points_t = points.transpose(0, 2, 1)
    valid = (jnp.arange(N)[None, :] < lengths[:, None]).astype(jnp.int32)
    valid = valid.reshape(B, 1, N)
    centers = _fps_centers(points_t, valid)
    lengths_pad = jnp.concatenate([lengths, jnp.zeros((4,), jnp.int32)])
    groups_flat, mask_flat = _sc_select(
        points_t, centers.reshape(B * _NUM_GROUPS, 4), lengths_pad)
    groups = groups_flat.reshape(B, _NUM_GROUPS, _GROUP_SIZE, 4)
    mask = mask_flat.reshape(B, _NUM_GROUPS, _GROUP_SIZE) != 0
    return groups, centers, mask
